# Initial kernel scaffold; baseline (speedup 1.0000x reference)
import jax
import jax.numpy as jnp
from jax.experimental import pallas as pl


def kernel(z, edge_index, edge_attr):
    raise NotImplementedError("write your pallas kernel here")


# speedup vs baseline: 1.9258x; 1.9258x over previous
import functools

import jax
import jax.numpy as jnp
from jax import lax
from jax.experimental import pallas as pl
from jax.experimental.pallas import tpu as pltpu
from jax.experimental.pallas import tpu_sc as plsc

N = 4096
D = 128
E = 131072
DE = 16
NN = N * N

NC, NS = 2, 16
NW = NC * NS
EPW = E // NW
COLS = 128
ROWS = EPW // COLS
GRP = COLS // 16

BM = 256


def _tc_body(zi_ref, zall_ref, out_ref):
    zz = lax.dot_general(
        zi_ref[...], zall_ref[...],
        (((1,), (1,)), ((), ())),
        preferred_element_type=jnp.float32,
    )
    out_ref[...] = 1.0 / (1.0 + jnp.exp(-zz))


def _dense_sigmoid(z):
    return pl.pallas_call(
        _tc_body,
        grid=(N // BM,),
        in_specs=[
            pl.BlockSpec((BM, D), lambda i: (i, 0)),
            pl.BlockSpec((N, D), lambda i: (0, 0)),
        ],
        out_specs=pl.BlockSpec((BM, N), lambda i: (i, 0)),
        out_shape=jax.ShapeDtypeStruct((N, N), jnp.float32),
    )(z, z)


_mesh = plsc.VectorSubcoreMesh(
    core_axis_name="c", subcore_axis_name="s", num_cores=NC, num_subcores=NS)


@functools.partial(
    pl.kernel,
    mesh=_mesh,
    compiler_params=pltpu.CompilerParams(needs_layout_passes=False),
    scratch_types=[
        pltpu.VMEM((EPW,), jnp.int32),
        pltpu.VMEM((EPW,), jnp.int32),
        pltpu.VMEM((ROWS, COLS), jnp.int32),
        pltpu.VMEM((ROWS, COLS), jnp.float32),
        pltpu.VMEM((EPW,), jnp.float32),
        pltpu.VMEM((EPW * DE,), jnp.float32),
        pltpu.SemaphoreType.DMA,
    ],
)
def _sc_fix(out_hbm, ei_hbm, ea_hbm, r_v, c_v, idx_v, y_v, en_v, ea_v, sem):
    wid = lax.axis_index("s") * NC + lax.axis_index("c")
    base = wid * EPW
    pltpu.sync_copy(ei_hbm.at[0, pl.ds(base, EPW)], r_v)
    pltpu.sync_copy(ei_hbm.at[1, pl.ds(base, EPW)], c_v)
    pltpu.sync_copy(ea_hbm.at[pl.ds(base * DE, EPW * DE)], ea_v)

    lanes = lax.iota(jnp.int32, 16)

    def build(j, _):
        for k in range(GRP):
            off = j * COLS + k * 16
            r = r_v[pl.ds(off, 16)]
            c = c_v[pl.ds(off, 16)]
            idx_v[j, pl.ds(k * 16, 16)] = r * N + c
            flat16 = (lanes + off) * DE
            acc = plsc.load_gather(ea_v, [flat16])
            for t in range(1, DE):
                acc = acc + plsc.load_gather(ea_v, [flat16 + t])
            en_v[pl.ds(off, 16)] = jnp.exp(acc * (-1.0 / DE))
        return 0

    lax.fori_loop(0, ROWS, build, 0)

    def gather(j, _):
        pltpu.async_copy(out_hbm.at[idx_v.at[j]], y_v.at[j], sem).wait()
        return 0

    lax.fori_loop(0, ROWS, gather, 0)

    def fix(j, _):
        for k in range(GRP):
            y = y_v[j, pl.ds(k * 16, 16)]
            en = en_v[pl.ds(j * COLS + k * 16, 16)]
            y_v[j, pl.ds(k * 16, 16)] = y / (y + en * (1.0 - y))
        return 0

    lax.fori_loop(0, ROWS, fix, 0)

    def scatter(j, _):
        pltpu.async_copy(y_v.at[j], out_hbm.at[idx_v.at[j]], sem).wait()
        return 0

    lax.fori_loop(0, ROWS, scatter, 0)


def kernel(z, edge_index, edge_attr):
    dense = _dense_sigmoid(z)
    ref = jax.new_ref(dense.reshape(NN))
    _sc_fix(ref, edge_index, edge_attr.reshape(E * DE))
    return ref[...].reshape(N, N)


# speedup vs baseline: 2.9381x; 1.5256x over previous
import functools

import jax
import jax.numpy as jnp
from jax import lax
from jax.experimental import pallas as pl
from jax.experimental.pallas import tpu as pltpu
from jax.experimental.pallas import tpu_sc as plsc

N = 4096
D = 128
E = 131072
DE = 16
NN = N * N

NC, NS = 2, 16
NW = NC * NS
EPW = E // NW
COLS = 128
ROWS = EPW // COLS
GRP = COLS // 16

BM = 256
EB = E // (N // BM)


def _tc_body(zi_ref, zall_ref, eat_ref, out_ref, en_ref):
    zz = lax.dot_general(
        zi_ref[...], zall_ref[...],
        (((1,), (1,)), ((), ())),
        preferred_element_type=jnp.float32,
    )
    out_ref[...] = (1.0 / (1.0 + jnp.exp(-zz))).reshape(BM * N)
    en_ref[...] = jnp.exp(jnp.sum(eat_ref[...], axis=0) * (-1.0 / DE))


def _dense_flat(z, eat):
    return pl.pallas_call(
        _tc_body,
        grid=(N // BM,),
        in_specs=[
            pl.BlockSpec((BM, D), lambda i: (i, 0)),
            pl.BlockSpec((N, D), lambda i: (0, 0)),
            pl.BlockSpec((DE, EB), lambda i: (0, i)),
        ],
        out_specs=[
            pl.BlockSpec((BM * N,), lambda i: (i,)),
            pl.BlockSpec((EB,), lambda i: (i,)),
        ],
        out_shape=[
            jax.ShapeDtypeStruct((NN,), jnp.float32),
            jax.ShapeDtypeStruct((E,), jnp.float32),
        ],
    )(z, z, eat)


def _retile_body(flat_ref, out_ref):
    out_ref[...] = flat_ref[...].reshape(BM, N)


def _retile(flat):
    return pl.pallas_call(
        _retile_body,
        grid=(N // BM,),
        in_specs=[pl.BlockSpec((BM * N,), lambda i: (i,))],
        out_specs=pl.BlockSpec((BM, N), lambda i: (i, 0)),
        out_shape=jax.ShapeDtypeStruct((N, N), jnp.float32),
    )(flat)


_mesh = plsc.VectorSubcoreMesh(
    core_axis_name="c", subcore_axis_name="s", num_cores=NC, num_subcores=NS)


@functools.partial(
    pl.kernel,
    mesh=_mesh,
    compiler_params=pltpu.CompilerParams(needs_layout_passes=False),
    scratch_types=[
        pltpu.VMEM((EPW,), jnp.int32),
        pltpu.VMEM((EPW,), jnp.int32),
        pltpu.VMEM((ROWS, COLS), jnp.int32),
        pltpu.VMEM((ROWS, COLS), jnp.float32),
        pltpu.VMEM((EPW,), jnp.float32),
        pltpu.SemaphoreType.DMA,
    ],
)
def _sc_fix(out_hbm, ei_hbm, en_hbm, r_v, c_v, idx_v, y_v, en_v, sem):
    wid = lax.axis_index("s") * NC + lax.axis_index("c")
    base = wid * EPW
    pltpu.sync_copy(ei_hbm.at[0, pl.ds(base, EPW)], r_v)
    pltpu.sync_copy(ei_hbm.at[1, pl.ds(base, EPW)], c_v)
    pltpu.sync_copy(en_hbm.at[pl.ds(base, EPW)], en_v)

    def build(j, _):
        for k in range(GRP):
            off = j * COLS + k * 16
            r = r_v[pl.ds(off, 16)]
            c = c_v[pl.ds(off, 16)]
            idx_v[j, pl.ds(k * 16, 16)] = r * N + c
        return 0

    lax.fori_loop(0, ROWS, build, 0)

    gathers = [
        pltpu.async_copy(out_hbm.at[idx_v.at[j]], y_v.at[j], sem)
        for j in range(ROWS)
    ]
    for g in gathers:
        g.wait()

    def fix(j, _):
        for k in range(GRP):
            y = y_v[j, pl.ds(k * 16, 16)]
            en = en_v[pl.ds(j * COLS + k * 16, 16)]
            y_v[j, pl.ds(k * 16, 16)] = y / (y + en * (1.0 - y))
        return 0

    lax.fori_loop(0, ROWS, fix, 0)

    scatters = [
        pltpu.async_copy(y_v.at[j], out_hbm.at[idx_v.at[j]], sem)
        for j in range(ROWS)
    ]
    for s in scatters:
        s.wait()


def kernel(z, edge_index, edge_attr):
    flat, en = _dense_flat(z, edge_attr.T)
    ref = jax.new_ref(flat)
    _sc_fix(ref, edge_index, en)
    return _retile(ref[...])


# speedup vs baseline: 2.9474x; 1.0032x over previous
import functools

import jax
import jax.numpy as jnp
from jax import lax
from jax.experimental import pallas as pl
from jax.experimental.pallas import tpu as pltpu
from jax.experimental.pallas import tpu_sc as plsc

N = 4096
D = 128
E = 131072
DE = 16
NN = N * N

NC, NS = 2, 16
NW = NC * NS
EPW = E // NW
COLS = 128
ROWS = EPW // COLS
GRP = COLS // 16

BM = 256
EB = E // (N // BM)


def _tc_body(zi_ref, zall_ref, eat_ref, out_ref, en_ref):
    zz = lax.dot_general(
        zi_ref[...], zall_ref[...],
        (((1,), (1,)), ((), ())),
        preferred_element_type=jnp.float32,
    )
    out_ref[...] = (1.0 / (1.0 + jnp.exp(-zz))).reshape(BM * N)
    en_ref[...] = jnp.exp(jnp.sum(eat_ref[...], axis=0) * (-1.0 / DE))


def _dense_flat(z, eat):
    return pl.pallas_call(
        _tc_body,
        grid=(N // BM,),
        in_specs=[
            pl.BlockSpec((BM, D), lambda i: (i, 0)),
            pl.BlockSpec((N, D), lambda i: (0, 0)),
            pl.BlockSpec((DE, EB), lambda i: (0, i)),
        ],
        out_specs=[
            pl.BlockSpec((BM * N,), lambda i: (i,)),
            pl.BlockSpec((EB,), lambda i: (i,)),
        ],
        out_shape=[
            jax.ShapeDtypeStruct((NN,), jnp.float32),
            jax.ShapeDtypeStruct((E,), jnp.float32),
        ],
    )(z, z, eat)


def _retile_body(flat_ref, out_ref):
    out_ref[...] = flat_ref[...].reshape(BM, N)


def _retile(flat):
    return pl.pallas_call(
        _retile_body,
        grid=(N // BM,),
        in_specs=[pl.BlockSpec((BM * N,), lambda i: (i,))],
        out_specs=pl.BlockSpec((BM, N), lambda i: (i, 0)),
        out_shape=jax.ShapeDtypeStruct((N, N), jnp.float32),
    )(flat)


_mesh = plsc.VectorSubcoreMesh(
    core_axis_name="c", subcore_axis_name="s", num_cores=NC, num_subcores=NS)


@functools.partial(
    pl.kernel,
    mesh=_mesh,
    compiler_params=pltpu.CompilerParams(needs_layout_passes=False),
    scratch_types=[
        pltpu.VMEM((EPW,), jnp.int32),
        pltpu.VMEM((EPW,), jnp.int32),
        pltpu.VMEM((EPW,), jnp.int32),
        pltpu.VMEM((EPW,), jnp.float32),
        pltpu.VMEM((EPW,), jnp.float32),
        pltpu.SemaphoreType.DMA,
    ],
)
def _sc_fix(out_hbm, ei_hbm, en_hbm, r_v, c_v, idx_v, y_v, en_v, sem):
    wid = lax.axis_index("s") * NC + lax.axis_index("c")
    base = wid * EPW
    pltpu.sync_copy(ei_hbm.at[0, pl.ds(base, EPW)], r_v)
    pltpu.sync_copy(ei_hbm.at[1, pl.ds(base, EPW)], c_v)
    pltpu.sync_copy(en_hbm.at[pl.ds(base, EPW)], en_v)

    def build(j, _):
        for k in range(GRP):
            off = j * COLS + k * 16
            r = r_v[pl.ds(off, 16)]
            c = c_v[pl.ds(off, 16)]
            idx_v[pl.ds(off, 16)] = r * N + c
        return 0

    lax.fori_loop(0, ROWS, build, 0)

    pltpu.async_copy(out_hbm.at[idx_v], y_v, sem).wait()

    def fix(j, _):
        for k in range(GRP):
            off = j * COLS + k * 16
            y = y_v[pl.ds(off, 16)]
            en = en_v[pl.ds(off, 16)]
            y_v[pl.ds(off, 16)] = y / (y + en * (1.0 - y))
        return 0

    lax.fori_loop(0, ROWS, fix, 0)

    pltpu.async_copy(y_v, out_hbm.at[idx_v], sem).wait()


def kernel(z, edge_index, edge_attr):
    flat, en = _dense_flat(z, edge_attr.T)
    ref = jax.new_ref(flat)
    _sc_fix(ref, edge_index, en)
    return _retile(ref[...])
